# Initial kernel scaffold; baseline (speedup 1.0000x reference)
import jax
import jax.numpy as jnp
from jax.experimental import pallas as pl


def kernel(hidden, attention_mask, W1, b1, W2, b2):
    raise NotImplementedError("write your pallas kernel here")


# speedup vs baseline: 1.0381x; 1.0381x over previous
import functools

import jax
import jax.numpy as jnp
import numpy as np
from jax.experimental import pallas as pl
from jax.experimental.pallas import tpu as pltpu

_PRIOR = 0.1
_B, _L, _D, _H = 4, 2048, 512, 1024
_ROWT = 1024
_ST = 256


def _mlp_kernel(x_ref, w1_ref, b1_ref, w2_ref, b2_ref, out_ref):
    h1 = jnp.maximum(
        jnp.dot(x_ref[...], w1_ref[...], preferred_element_type=jnp.float32)
        + b1_ref[...], 0.0)
    out_ref[...] = (
        jnp.dot(h1, w2_ref[...], preferred_element_type=jnp.float32)
        + b2_ref[...])


def _boundary_kernel(logits_ref, mask_ref, seg_ref, short_ref, counts_ref,
                     loss_ref, nb_ref, tp_ref):
    B, L = logits_ref.shape
    logits = logits_ref[...]
    mask = mask_ref[...]
    probs = jax.nn.sigmoid(logits)
    hard = jnp.where(probs > 0.5, 1.0, 0.0) * mask
    n = jnp.sum(mask, axis=1, keepdims=True)
    col = jax.lax.broadcasted_iota(jnp.int32, (B, L), 1).astype(jnp.float32)
    last_real = jnp.where((col == n - 1.0) & (n < float(L)), 1.0, 0.0)
    hard = jnp.maximum(hard, last_real)
    cum = hard
    for k in range(11):
        s = 1 << k
        rolled = jnp.roll(cum, s, axis=1)
        cum = cum + jnp.where(col >= float(s), rolled, 0.0)
    seg_ref[...] = cum - hard
    counts = jnp.sum(hard, axis=1, keepdims=True)
    counts_ref[...] = counts
    short_ref[...] = jnp.where(col < counts, 1.0, 0.0)
    nb_ref[...] = jnp.sum(counts).reshape(1, 1)
    tp_ref[...] = jnp.sum(n).reshape(1, 1)
    kval = col + 1.0
    logk = jnp.log(kval)

    def logfact(m):
        return jnp.sum(jnp.where((kval >= 2.0) & (kval <= m), logk, 0.0),
                       axis=1, keepdims=True)

    logprob = (logfact(n) - logfact(counts) - logfact(n - counts)
               + counts * np.log(_PRIOR) + (n - counts) * np.log1p(-_PRIOR))
    loss_ref[...] = (10.0 * jnp.mean(-(logprob / n))).reshape(1, 1)


def _pool_kernel(seg_ref, hid_ref, pe_ref, out_ref):
    st = pl.program_id(1)
    L = seg_ref.shape[2]
    seg = seg_ref[0]
    srow = (jax.lax.broadcasted_iota(jnp.int32, (_ST, L), 0).astype(jnp.float32)
            + jnp.float32(_ST) * st.astype(jnp.float32))
    onehot = jnp.where(seg == srow, 1.0, 0.0)
    cnt = jnp.sum(onehot, axis=1, keepdims=True)
    acc = jnp.dot(onehot, hid_ref[0], preferred_element_type=jnp.float32)
    out_ref[0] = acc / (cnt + 1e-9) + pe_ref[...]


def _pos_emb(S, D):
    pos = jnp.arange(S, dtype=jnp.float32)[:, None]
    i = jnp.arange(0, D, 2, dtype=jnp.float32)[None, :]
    div = jnp.exp(-(jnp.log(10000.0)) * i / D)
    pe = jnp.zeros((S, D), dtype=jnp.float32)
    pe = pe.at[:, 0::2].set(jnp.sin(pos * div))
    pe = pe.at[:, 1::2].set(jnp.cos(pos * div))
    return pe


@jax.jit
def kernel(hidden, attention_mask, W1, b1, W2, b2):
    B, L, D = hidden.shape
    H = W1.shape[1]
    f32 = jnp.float32

    logits = pl.pallas_call(
        _mlp_kernel,
        grid=(B * L // _ROWT,),
        in_specs=[
            pl.BlockSpec((_ROWT, D), lambda i: (i, 0)),
            pl.BlockSpec((D, H), lambda i: (0, 0)),
            pl.BlockSpec((1, H), lambda i: (0, 0)),
            pl.BlockSpec((H, 1), lambda i: (0, 0)),
            pl.BlockSpec((1, 1), lambda i: (0, 0)),
        ],
        out_specs=pl.BlockSpec((_ROWT, 1), lambda i: (i, 0)),
        out_shape=jax.ShapeDtypeStruct((B * L, 1), f32),
    )(hidden.reshape(B * L, D), W1, b1.reshape(1, H), W2, b2.reshape(1, 1))
    logits = logits.reshape(B, L)

    seg, short_mask, counts, loss, nb, tp = pl.pallas_call(
        _boundary_kernel,
        out_shape=(
            jax.ShapeDtypeStruct((B, L), f32),
            jax.ShapeDtypeStruct((B, L), f32),
            jax.ShapeDtypeStruct((B, 1), f32),
            jax.ShapeDtypeStruct((1, 1), f32),
            jax.ShapeDtypeStruct((1, 1), f32),
            jax.ShapeDtypeStruct((1, 1), f32),
        ),
    )(logits, attention_mask)

    pe = _pos_emb(L, D)
    pooled = pl.pallas_call(
        _pool_kernel,
        grid=(B, L // _ST),
        in_specs=[
            pl.BlockSpec((1, 1, L), lambda b, s: (b, 0, 0)),
            pl.BlockSpec((1, L, D), lambda b, s: (b, 0, 0)),
            pl.BlockSpec((_ST, D), lambda b, s: (s, 0)),
        ],
        out_specs=pl.BlockSpec((1, _ST, D), lambda b, s: (b, s, 0)),
        out_shape=jax.ShapeDtypeStruct((B, L, D), f32),
    )(seg.reshape(B, 1, L), hidden, pe)

    return (pooled, loss[0, 0], nb[0, 0], tp[0, 0], short_mask)
